# baseline (speedup 1.0000x reference)
import functools

import jax
import jax.numpy as jnp
from jax import lax
from jax.experimental import pallas as pl
from jax.experimental.pallas import tpu as pltpu
from jax.experimental.pallas import tpu_sc as plsc

NC = 2
NS = 16
NW = NC * NS
LANES = 16
CHUNK = 128
NBUF = 5


def _pad_sizes(n_edges, n_nodes):
    ct = -(-n_edges // (NS * CHUNK))
    ct = -(-ct // NBUF) * NBUF
    e_pad = NS * ct * CHUNK
    m = NS * 8
    npad = -(-(n_nodes + 1) // m) * m
    return ct, e_pad, npad


def _degree_body(n_pad, e_w, dst_hbm, out_hbm, dst_v, acc):
    c = lax.axis_index("c")
    s = lax.axis_index("s")
    w = c * NS + s
    ones16 = jnp.full((LANES,), 1.0, jnp.float32)
    zero16 = jnp.full((LANES,), 0.0, jnp.float32)
    pltpu.sync_copy(dst_hbm.at[w], dst_v)

    def zbody(i, carry):
        for u in range(4):
            acc[pl.ds((4 * i + u) * LANES, LANES)] = zero16
        return carry

    lax.fori_loop(0, n_pad // (4 * LANES), zbody, 0)

    def ebody(i, carry):
        for u in range(4):
            di = dst_v[pl.ds((4 * i + u) * LANES, LANES)]
            plsc.addupdate_scatter(acc, [di], ones16)
        return carry

    lax.fori_loop(0, e_w // (4 * LANES), ebody, 0)
    pltpu.sync_copy(acc, out_hbm.at[w])


def _make_degree_kernel(n_pad, e_w):
    mesh = plsc.VectorSubcoreMesh(core_axis_name="c", subcore_axis_name="s")
    body = functools.partial(_degree_body, n_pad, e_w)
    return pl.kernel(
        body,
        out_type=jax.ShapeDtypeStruct((NW, n_pad), jnp.float32),
        mesh=mesh,
        compiler_params=pltpu.CompilerParams(needs_layout_passes=False),
        scratch_types=[
            pltpu.VMEM((e_w,), jnp.int32),
            pltpu.VMEM((n_pad,), jnp.float32),
        ],
    )


def _agg_body(n_pad, n_chunks, half,
              hs_hbm, src_hbm, dst_hbm, zeros_hbm, out_hbm,
              src_v, dst_v, b0, b1, b2, b3, b4, acc, sem_g, sem_s):
    c = lax.axis_index("c")
    s = lax.axis_index("s")
    rows_per_tile = n_pad // NS
    e_t = n_chunks * CHUNK
    bufs = (b0, b1, b2, b3, b4)
    hs_c = hs_hbm.at[c]

    z_cp = pltpu.async_copy(
        zeros_hbm, acc.at[pl.ds(s * rows_per_tile, rows_per_tile)], sem_s)
    s_cp = pltpu.async_copy(src_hbm.at[pl.ds(s * e_t, e_t)], src_v, sem_g)
    d_cp = pltpu.async_copy(dst_hbm.at[pl.ds(s * n_chunks, n_chunks)],
                            dst_v, sem_g)
    s_cp.wait()
    d_cp.wait()
    z_cp.wait()
    plsc.subcore_barrier()

    def gather(j, buf):
        pltpu.async_copy(
            hs_c.at[src_v.at[pl.ds(j * CHUNK, CHUNK)]], buf, sem_g)

    for t in range(NBUF - 1):
        gather(t, bufs[t])

    def wait_scatter(buf):
        pltpu.make_async_copy(buf, acc.at[pl.ds(0, CHUNK)], sem_s).wait()

    def ebody(i, carry):
        j0 = NBUF * i
        for t in range(NBUF):
            j = j0 + t
            buf = bufs[t]
            prev = bufs[(t + NBUF - 1) % NBUF]
            pltpu.make_async_copy(hs_c, buf, sem_g).wait()
            if t == 0:
                @pl.when(i > 0)
                def _():
                    wait_scatter(prev)
            else:
                wait_scatter(prev)

            @pl.when(j + NBUF - 1 < n_chunks)
            def _():
                gather(j + NBUF - 1, prev)

            pltpu.async_copy(buf, acc.at[dst_v.at[j]], sem_s, add=True)
        return carry

    lax.fori_loop(0, n_chunks // NBUF, ebody, 0)
    wait_scatter(bufs[(NBUF - 1) % NBUF])
    plsc.subcore_barrier()
    pltpu.sync_copy(
        acc.at[pl.ds(s * rows_per_tile, rows_per_tile)],
        out_hbm.at[c, pl.ds(s * rows_per_tile, rows_per_tile)],
    )


def _make_agg_kernel(n_pad, n_chunks, half):
    mesh = plsc.VectorSubcoreMesh(core_axis_name="c", subcore_axis_name="s")
    body = functools.partial(_agg_body, n_pad, n_chunks, half)
    rowbuf = pltpu.VMEM((CHUNK, half), jnp.float32)
    return pl.kernel(
        body,
        out_type=jax.ShapeDtypeStruct((NC, n_pad, half), jnp.float32),
        mesh=mesh,
        compiler_params=pltpu.CompilerParams(
            needs_layout_passes=False, use_tc_tiling_on_sc=False),
        scratch_types=[
            pltpu.VMEM((NS * n_chunks * CHUNK // NS,), jnp.int32),
            pltpu.VMEM((n_chunks, CHUNK), jnp.int32),
            rowbuf, rowbuf, rowbuf, rowbuf, rowbuf,
            pltpu.VMEM_SHARED((n_pad, half), jnp.float32),
            pltpu.SemaphoreType.DMA,
            pltpu.SemaphoreType.DMA,
        ],
    )


def _scalar_agg_body(n_nodes, n_pad, e_w,
                     z_hbm, src_hbm, dst_hbm, out_hbm,
                     z_v, src_v, dst_v, acc):
    zero16 = jnp.full((LANES,), 0.0, jnp.float32)
    c = lax.axis_index("c")
    s = lax.axis_index("s")
    w = c * NS + s
    pltpu.sync_copy(z_hbm, z_v)
    pltpu.sync_copy(src_hbm.at[w], src_v)
    pltpu.sync_copy(dst_hbm.at[w], dst_v)

    def zbody(i, carry):
        for u in range(4):
            acc[pl.ds((4 * i + u) * LANES, LANES)] = zero16
        return carry

    lax.fori_loop(0, n_pad // (4 * LANES), zbody, 0)

    def ebody(i, carry):
        for u in range(4):
            si = src_v[pl.ds((4 * i + u) * LANES, LANES)]
            di = dst_v[pl.ds((4 * i + u) * LANES, LANES)]
            vals = plsc.load_gather(z_v, [si])
            plsc.addupdate_scatter(acc, [di], vals)
        return carry

    lax.fori_loop(0, e_w // (4 * LANES), ebody, 0)
    pltpu.sync_copy(acc, out_hbm.at[w])


def _make_scalar_agg_kernel(n_nodes, n_pad, e_w):
    mesh = plsc.VectorSubcoreMesh(core_axis_name="c", subcore_axis_name="s")
    body = functools.partial(_scalar_agg_body, n_nodes, n_pad, e_w)
    return pl.kernel(
        body,
        out_type=jax.ShapeDtypeStruct((NW, n_pad), jnp.float32),
        mesh=mesh,
        compiler_params=pltpu.CompilerParams(needs_layout_passes=False),
        scratch_types=[
            pltpu.VMEM((n_nodes,), jnp.float32),
            pltpu.VMEM((e_w,), jnp.int32),
            pltpu.VMEM((e_w,), jnp.int32),
            pltpu.VMEM((n_pad,), jnp.float32),
        ],
    )


def _dinv_from_partials(degp):
    deg = jnp.sum(degp, axis=0) + 1.0
    return lax.rsqrt(deg)


def _split_store(out_ref, r, half):
    out_ref[0] = r[:, :half]
    out_ref[1] = r[:, half:]


def _tc_mm_body(x_ref, w_ref, out_ref):
    out_ref[...] = jnp.dot(x_ref[...], w_ref[...],
                           preferred_element_type=jnp.float32)


def _tc_scale_body(half, degp_ref, mm_ref, hs_ref):
    dinv = _dinv_from_partials(degp_ref[...])
    _split_store(hs_ref, mm_ref[...] * dinv[:, None], half)


def _tc_mid_body(half, split, degp_ref, agg_ref, hsp_ref, b_ref, w_ref,
                 out_ref):
    dinv = _dinv_from_partials(degp_ref[...])
    a = (jnp.concatenate([agg_ref[0], agg_ref[1]], axis=-1)
         + jnp.concatenate([hsp_ref[0], hsp_ref[1]], axis=-1))
    h = jnp.maximum(a * dinv[:, None] + b_ref[...], 0.0)
    r = (jnp.dot(h, w_ref[...], preferred_element_type=jnp.float32)
         * dinv[:, None])
    if split:
        _split_store(out_ref, r, half)
    else:
        out_ref[...] = r


def _tc_final_body(degp_ref, accs_ref, z_ref, b_ref, out_ref):
    dinv = _dinv_from_partials(degp_ref[...])
    a = jnp.sum(accs_ref[...], axis=0) + z_ref[...][:, 0]
    out_ref[...] = (dinv * a)[:, None] + b_ref[...]


def kernel(x, edge_index, W1, b1, W2, b2, W3, b3):
    n_nodes, in_dim = x.shape
    hid = W1.shape[1]
    half = hid // 2
    n_edges = edge_index.shape[1]
    n_chunks, e_pad, n_pad = _pad_sizes(n_edges, n_nodes)
    e_w = e_pad // NW
    pad = e_pad - n_edges

    src = jnp.concatenate([edge_index[0], jnp.zeros((pad,), jnp.int32)])
    dst = jnp.concatenate(
        [edge_index[1], jnp.full((pad,), n_nodes, jnp.int32)])
    dst2d = dst.reshape(e_pad // CHUNK, CHUNK)
    src1d = src.reshape(NW, e_w)
    dst1d = dst.reshape(NW, e_w)
    zeros_blk = jnp.zeros((n_pad // NS, half), jnp.float32)

    degree_k = _make_degree_kernel(n_pad, e_w)
    agg_k = _make_agg_kernel(n_pad, n_chunks, half)
    scalar_k = _make_scalar_agg_kernel(n_nodes, n_pad, e_w)

    mm1 = pl.pallas_call(
        _tc_mm_body,
        out_shape=jax.ShapeDtypeStruct((n_nodes, hid), jnp.float32),
    )(x, W1)
    degp = degree_k(dst1d)[:, :n_nodes]

    tc_final = pl.pallas_call(
        _tc_final_body,
        out_shape=jax.ShapeDtypeStruct((n_nodes, 1), jnp.float32),
    )

    hs1 = pl.pallas_call(
        functools.partial(_tc_scale_body, half),
        out_shape=jax.ShapeDtypeStruct((NC, n_nodes, half), jnp.float32),
    )(degp, mm1)
    agg1 = agg_k(hs1, src, dst2d, zeros_blk)[:, :n_nodes, :]
    hs2 = pl.pallas_call(
        functools.partial(_tc_mid_body, half, True),
        out_shape=jax.ShapeDtypeStruct((NC, n_nodes, half), jnp.float32),
    )(degp, agg1, hs1, b1.reshape(1, hid), W2)
    agg2 = agg_k(hs2, src, dst2d, zeros_blk)[:, :n_nodes, :]
    z = pl.pallas_call(
        functools.partial(_tc_mid_body, half, False),
        out_shape=jax.ShapeDtypeStruct((n_nodes, 1), jnp.float32),
    )(degp, agg2, hs2, b2.reshape(1, hid), W3)
    accs = scalar_k(z.reshape(n_nodes), src1d, dst1d)[:, :n_nodes]
    out = tc_final(degp, accs, z, b3.reshape(1, 1))
    return out


# speedup vs baseline: 1.1109x; 1.1109x over previous
import functools

import jax
import jax.numpy as jnp
from jax import lax
from jax.experimental import pallas as pl
from jax.experimental.pallas import tpu as pltpu
from jax.experimental.pallas import tpu_sc as plsc

NC = 2
NS = 16
NW = NC * NS
LANES = 16
CHUNK = 128
NBUF = 5


def _pad_sizes(n_edges, n_nodes):
    ct = -(-n_edges // (NS * CHUNK))
    ct = -(-ct // NBUF) * NBUF
    e_pad = NS * ct * CHUNK
    m = NS * 8
    npad = -(-(n_nodes + 1) // m) * m
    return ct, e_pad, npad


def _degree_body(n_pad, e_w, dst_hbm, out_hbm, dst_v, acc):
    c = lax.axis_index("c")
    s = lax.axis_index("s")
    w = c * NS + s
    ones16 = jnp.full((LANES,), 1.0, jnp.float32)
    zero16 = jnp.full((LANES,), 0.0, jnp.float32)
    pltpu.sync_copy(dst_hbm.at[w], dst_v)

    def zbody(i, carry):
        for u in range(4):
            acc[pl.ds((4 * i + u) * LANES, LANES)] = zero16
        return carry

    lax.fori_loop(0, n_pad // (4 * LANES), zbody, 0)

    def ebody(i, carry):
        for u in range(4):
            di = dst_v[pl.ds((4 * i + u) * LANES, LANES)]
            plsc.addupdate_scatter(acc, [di], ones16)
        return carry

    lax.fori_loop(0, e_w // (4 * LANES), ebody, 0)
    pltpu.sync_copy(acc, out_hbm.at[w])


def _make_degree_kernel(n_pad, e_w):
    mesh = plsc.VectorSubcoreMesh(core_axis_name="c", subcore_axis_name="s")
    body = functools.partial(_degree_body, n_pad, e_w)
    return pl.kernel(
        body,
        out_type=jax.ShapeDtypeStruct((NW, n_pad), jnp.float32),
        mesh=mesh,
        compiler_params=pltpu.CompilerParams(needs_layout_passes=False),
        scratch_types=[
            pltpu.VMEM((e_w,), jnp.int32),
            pltpu.VMEM((n_pad,), jnp.float32),
        ],
    )


def _agg_body(n_pad, n_chunks, half,
              hs_hbm, src_hbm, dst_hbm, zeros_hbm, out_hbm,
              src_v, dst_v, b0, b1, b2, b3, b4, acc, sem_g, sem_s):
    c = lax.axis_index("c")
    s = lax.axis_index("s")
    rows_per_tile = n_pad // NS
    e_t = n_chunks * CHUNK
    bufs = (b0, b1, b2, b3, b4)
    hs_c = hs_hbm.at[c]

    z_cp = pltpu.async_copy(
        zeros_hbm, acc.at[pl.ds(s * rows_per_tile, rows_per_tile)], sem_s)
    s_cp = pltpu.async_copy(src_hbm.at[pl.ds(s * e_t, e_t)], src_v, sem_g)
    d_cp = pltpu.async_copy(dst_hbm.at[pl.ds(s * n_chunks, n_chunks)],
                            dst_v, sem_g)
    s_cp.wait()
    d_cp.wait()
    z_cp.wait()
    plsc.subcore_barrier()

    def gather(j, buf):
        pltpu.async_copy(
            hs_c.at[src_v.at[pl.ds(j * CHUNK, CHUNK)]], buf, sem_g)

    for t in range(NBUF - 1):
        gather(t, bufs[t])

    def wait_scatter(buf):
        pltpu.make_async_copy(buf, acc.at[pl.ds(0, CHUNK)], sem_s).wait()

    def ebody(i, carry):
        j0 = NBUF * i
        for t in range(NBUF):
            j = j0 + t
            buf = bufs[t]
            prev = bufs[(t + NBUF - 1) % NBUF]
            pltpu.make_async_copy(hs_c, buf, sem_g).wait()
            if t == 0:
                @pl.when(i > 0)
                def _():
                    wait_scatter(prev)
            else:
                wait_scatter(prev)

            @pl.when(j + NBUF - 1 < n_chunks)
            def _():
                gather(j + NBUF - 1, prev)

            pltpu.async_copy(buf, acc.at[dst_v.at[j]], sem_s, add=True)
        return carry

    lax.fori_loop(0, n_chunks // NBUF, ebody, 0)
    wait_scatter(bufs[(NBUF - 1) % NBUF])
    plsc.subcore_barrier()
    pltpu.sync_copy(
        acc.at[pl.ds(s * rows_per_tile, rows_per_tile)],
        out_hbm.at[c, pl.ds(s * rows_per_tile, rows_per_tile)],
    )


def _make_agg_kernel(n_pad, n_chunks, half):
    mesh = plsc.VectorSubcoreMesh(core_axis_name="c", subcore_axis_name="s")
    body = functools.partial(_agg_body, n_pad, n_chunks, half)
    rowbuf = pltpu.VMEM((CHUNK, half), jnp.float32)
    return pl.kernel(
        body,
        out_type=jax.ShapeDtypeStruct((NC, n_pad, half), jnp.float32),
        mesh=mesh,
        compiler_params=pltpu.CompilerParams(
            needs_layout_passes=False, use_tc_tiling_on_sc=False),
        scratch_types=[
            pltpu.VMEM((NS * n_chunks * CHUNK // NS,), jnp.int32),
            pltpu.VMEM((n_chunks, CHUNK), jnp.int32),
            rowbuf, rowbuf, rowbuf, rowbuf, rowbuf,
            pltpu.VMEM_SHARED((n_pad, half), jnp.float32),
            pltpu.SemaphoreType.DMA,
            pltpu.SemaphoreType.DMA,
        ],
    )


def _scalar_agg_body(n_pad, e_w,
                     z_hbm, src_hbm, dst_hbm, out_hbm,
                     z_v, src_v, dst_v, acc):
    zero16 = jnp.full((LANES,), 0.0, jnp.float32)
    c = lax.axis_index("c")
    s = lax.axis_index("s")
    w = c * NS + s
    pltpu.sync_copy(z_hbm, z_v)
    pltpu.sync_copy(src_hbm.at[w], src_v)
    pltpu.sync_copy(dst_hbm.at[w], dst_v)

    def zbody(i, carry):
        for u in range(4):
            acc[pl.ds((4 * i + u) * LANES, LANES)] = zero16
        return carry

    lax.fori_loop(0, n_pad // (4 * LANES), zbody, 0)

    def ebody(i, carry):
        for u in range(4):
            si = src_v[pl.ds((4 * i + u) * LANES, LANES)]
            di = dst_v[pl.ds((4 * i + u) * LANES, LANES)]
            vals = plsc.load_gather(z_v, [si])
            plsc.addupdate_scatter(acc, [di], vals)
        return carry

    lax.fori_loop(0, e_w // (4 * LANES), ebody, 0)
    pltpu.sync_copy(acc, out_hbm.at[w])


def _make_scalar_agg_kernel(n_pad, e_w):
    mesh = plsc.VectorSubcoreMesh(core_axis_name="c", subcore_axis_name="s")
    body = functools.partial(_scalar_agg_body, n_pad, e_w)
    return pl.kernel(
        body,
        out_type=jax.ShapeDtypeStruct((NW, n_pad), jnp.float32),
        mesh=mesh,
        compiler_params=pltpu.CompilerParams(needs_layout_passes=False),
        scratch_types=[
            pltpu.VMEM((n_pad,), jnp.float32),
            pltpu.VMEM((e_w,), jnp.int32),
            pltpu.VMEM((e_w,), jnp.int32),
            pltpu.VMEM((n_pad,), jnp.float32),
        ],
    )


def _dinv_from_partials(degp):
    deg = jnp.sum(degp, axis=0) + 1.0
    return lax.rsqrt(deg)


def _split_store(out_ref, r, half):
    out_ref[0] = r[:, :half]
    out_ref[1] = r[:, half:]


def _tc_first_body(half, degp_ref, x_ref, w_ref, hs_ref):
    dinv = _dinv_from_partials(degp_ref[...])
    h = jnp.dot(x_ref[...], w_ref[...], preferred_element_type=jnp.float32)
    _split_store(hs_ref, h * dinv[:, None], half)


def _tc_mid_body(half, split, degp_ref, agg_ref, hsp_ref, b_ref, w_ref,
                 out_ref):
    dinv = _dinv_from_partials(degp_ref[...])
    a = (jnp.concatenate([agg_ref[0], agg_ref[1]], axis=-1)
         + jnp.concatenate([hsp_ref[0], hsp_ref[1]], axis=-1))
    h = jnp.maximum(a * dinv[:, None] + b_ref[...], 0.0)
    r = (jnp.dot(h, w_ref[...], preferred_element_type=jnp.float32)
         * dinv[:, None])
    if split:
        _split_store(out_ref, r, half)
    else:
        out_ref[...] = r


def _tc_final_body(degp_ref, accs_ref, z_ref, b_ref, out_ref):
    dinv = _dinv_from_partials(degp_ref[...])
    a = jnp.sum(accs_ref[...], axis=0) + z_ref[...][:, 0]
    out_ref[...] = (dinv * a)[:, None] + b_ref[...]


def kernel(x, edge_index, W1, b1, W2, b2, W3, b3):
    n_nodes, in_dim = x.shape
    hid = W1.shape[1]
    half = hid // 2
    n_edges = edge_index.shape[1]
    n_chunks, e_pad, n_pad = _pad_sizes(n_edges, n_nodes)
    e_w = e_pad // NW
    pad = e_pad - n_edges

    src = jnp.concatenate([edge_index[0], jnp.zeros((pad,), jnp.int32)])
    dst = jnp.concatenate(
        [edge_index[1], jnp.full((pad,), n_nodes, jnp.int32)])
    dst2d = dst.reshape(e_pad // CHUNK, CHUNK)
    src1d = src.reshape(NW, e_w)
    dst1d = dst.reshape(NW, e_w)
    zeros_blk = jnp.zeros((n_pad // NS, half), jnp.float32)

    degree_k = _make_degree_kernel(n_pad, e_w)
    agg_k = _make_agg_kernel(n_pad, n_chunks, half)
    scalar_k = _make_scalar_agg_kernel(n_pad, e_w)

    x_pad = jnp.concatenate(
        [x, jnp.zeros((n_pad - n_nodes, in_dim), jnp.float32)])
    degp = degree_k(dst1d)

    hs1 = pl.pallas_call(
        functools.partial(_tc_first_body, half),
        out_shape=jax.ShapeDtypeStruct((NC, n_pad, half), jnp.float32),
    )(degp, x_pad, W1)
    agg1 = agg_k(hs1, src, dst2d, zeros_blk)
    hs2 = pl.pallas_call(
        functools.partial(_tc_mid_body, half, True),
        out_shape=jax.ShapeDtypeStruct((NC, n_pad, half), jnp.float32),
    )(degp, agg1, hs1, b1.reshape(1, hid), W2)
    agg2 = agg_k(hs2, src, dst2d, zeros_blk)
    z = pl.pallas_call(
        functools.partial(_tc_mid_body, half, False),
        out_shape=jax.ShapeDtypeStruct((n_pad, 1), jnp.float32),
    )(degp, agg2, hs2, b2.reshape(1, hid), W3)
    accs = scalar_k(z.reshape(n_pad), src1d, dst1d)
    out = pl.pallas_call(
        _tc_final_body,
        out_shape=jax.ShapeDtypeStruct((n_pad, 1), jnp.float32),
    )(degp, accs, z, b3.reshape(1, 1))
    return out[:n_nodes]


# speedup vs baseline: 1.1919x; 1.0729x over previous
import functools

import jax
import jax.numpy as jnp
from jax import lax
from jax.experimental import pallas as pl
from jax.experimental.pallas import tpu as pltpu
from jax.experimental.pallas import tpu_sc as plsc

NC = 2
NS = 16
NW = NC * NS
LANES = 16
CHUNK = 128
NBUF = 5


def _pad_sizes(n_edges, n_nodes):
    ct = -(-n_edges // (NS * CHUNK))
    ct = -(-ct // NBUF) * NBUF
    e_pad = NS * ct * CHUNK
    m = NS * 8
    npad = -(-(n_nodes + 1) // m) * m
    return ct, e_pad, npad


def _degree_body(n_pad, e_w, dst_hbm, out_hbm, dst_v, acc):
    c = lax.axis_index("c")
    s = lax.axis_index("s")
    w = c * NS + s
    ones16 = jnp.full((LANES,), 1.0, jnp.float32)
    zero16 = jnp.full((LANES,), 0.0, jnp.float32)
    pltpu.sync_copy(dst_hbm.at[w], dst_v)

    def zbody(i, carry):
        for u in range(4):
            acc[pl.ds((4 * i + u) * LANES, LANES)] = zero16
        return carry

    lax.fori_loop(0, n_pad // (4 * LANES), zbody, 0)

    def ebody(i, carry):
        for u in range(4):
            di = dst_v[pl.ds((4 * i + u) * LANES, LANES)]
            plsc.addupdate_scatter(acc, [di], ones16)
        return carry

    lax.fori_loop(0, e_w // (4 * LANES), ebody, 0)
    pltpu.sync_copy(acc, out_hbm.at[w])


def _make_degree_kernel(n_pad, e_w):
    mesh = plsc.VectorSubcoreMesh(core_axis_name="c", subcore_axis_name="s")
    body = functools.partial(_degree_body, n_pad, e_w)
    return pl.kernel(
        body,
        out_type=jax.ShapeDtypeStruct((NW, n_pad), jnp.float32),
        mesh=mesh,
        compiler_params=pltpu.CompilerParams(needs_layout_passes=False),
        scratch_types=[
            pltpu.VMEM((e_w,), jnp.int32),
            pltpu.VMEM((n_pad,), jnp.float32),
        ],
    )


def _agg_body(n_pad, n_chunks, half,
              hs_hbm, src_hbm, dst_hbm, zeros_hbm, out_hbm,
              src_v, dst_v, b0, b1, b2, b3, b4, acc, sem_g, sem_s):
    c = lax.axis_index("c")
    s = lax.axis_index("s")
    rows_per_tile = n_pad // NS
    e_t = n_chunks * CHUNK
    bufs = (b0, b1, b2, b3, b4)
    hs_c = hs_hbm.at[c]

    z_cp = pltpu.async_copy(
        zeros_hbm, acc.at[pl.ds(s * rows_per_tile, rows_per_tile)], sem_s)
    s_cp = pltpu.async_copy(src_hbm.at[pl.ds(s * e_t, e_t)], src_v, sem_g)
    d_cp = pltpu.async_copy(dst_hbm.at[pl.ds(s * n_chunks, n_chunks)],
                            dst_v, sem_g)
    s_cp.wait()
    d_cp.wait()
    z_cp.wait()
    plsc.subcore_barrier()

    def gather(j, buf):
        pltpu.async_copy(
            hs_c.at[src_v.at[pl.ds(j * CHUNK, CHUNK)]], buf, sem_g)

    for t in range(NBUF - 1):
        gather(t, bufs[t])

    def wait_scatter(buf):
        pltpu.make_async_copy(buf, acc.at[pl.ds(0, CHUNK)], sem_s).wait()

    def ebody(i, carry):
        j0 = NBUF * i
        for t in range(NBUF):
            j = j0 + t
            buf = bufs[t]
            prev = bufs[(t + NBUF - 1) % NBUF]
            pltpu.make_async_copy(hs_c, buf, sem_g).wait()
            if t == 0:
                @pl.when(i > 0)
                def _():
                    wait_scatter(prev)
            else:
                wait_scatter(prev)

            @pl.when(j + NBUF - 1 < n_chunks)
            def _():
                gather(j + NBUF - 1, prev)

            pltpu.async_copy(buf, acc.at[dst_v.at[j]], sem_s, add=True)
        return carry

    lax.fori_loop(0, n_chunks // NBUF, ebody, 0)
    wait_scatter(bufs[(NBUF - 1) % NBUF])
    plsc.subcore_barrier()
    pltpu.sync_copy(
        acc.at[pl.ds(s * rows_per_tile, rows_per_tile)],
        out_hbm.at[c, pl.ds(s * rows_per_tile, rows_per_tile)],
    )


def _make_agg_kernel(n_pad, n_chunks, half):
    mesh = plsc.VectorSubcoreMesh(core_axis_name="c", subcore_axis_name="s")
    body = functools.partial(_agg_body, n_pad, n_chunks, half)
    rowbuf = pltpu.VMEM((CHUNK, half), jnp.float32)
    return pl.kernel(
        body,
        out_type=jax.ShapeDtypeStruct((NC, n_pad, half), jnp.float32),
        mesh=mesh,
        compiler_params=pltpu.CompilerParams(
            needs_layout_passes=False, use_tc_tiling_on_sc=False),
        scratch_types=[
            pltpu.VMEM((NS * n_chunks * CHUNK // NS,), jnp.int32),
            pltpu.VMEM((n_chunks, CHUNK), jnp.int32),
            rowbuf, rowbuf, rowbuf, rowbuf, rowbuf,
            pltpu.VMEM_SHARED((n_pad, half), jnp.float32),
            pltpu.SemaphoreType.DMA,
            pltpu.SemaphoreType.DMA,
        ],
    )


def _scalar_agg_body(n_nodes, n_pad, e_w,
                     z_hbm, src_hbm, dst_hbm, out_hbm,
                     z_v, src_v, dst_v, acc):
    zero16 = jnp.full((LANES,), 0.0, jnp.float32)
    c = lax.axis_index("c")
    s = lax.axis_index("s")
    w = c * NS + s
    pltpu.sync_copy(z_hbm, z_v)
    pltpu.sync_copy(src_hbm.at[w], src_v)
    pltpu.sync_copy(dst_hbm.at[w], dst_v)

    def zbody(i, carry):
        for u in range(4):
            acc[pl.ds((4 * i + u) * LANES, LANES)] = zero16
        return carry

    lax.fori_loop(0, n_pad // (4 * LANES), zbody, 0)

    def ebody(i, carry):
        for u in range(4):
            si = src_v[pl.ds((4 * i + u) * LANES, LANES)]
            di = dst_v[pl.ds((4 * i + u) * LANES, LANES)]
            vals = plsc.load_gather(z_v, [si])
            plsc.addupdate_scatter(acc, [di], vals)
        return carry

    lax.fori_loop(0, e_w // (4 * LANES), ebody, 0)
    pltpu.sync_copy(acc, out_hbm.at[w])


def _make_scalar_agg_kernel(n_nodes, n_pad, e_w):
    mesh = plsc.VectorSubcoreMesh(core_axis_name="c", subcore_axis_name="s")
    body = functools.partial(_scalar_agg_body, n_nodes, n_pad, e_w)
    return pl.kernel(
        body,
        out_type=jax.ShapeDtypeStruct((NW, n_pad), jnp.float32),
        mesh=mesh,
        compiler_params=pltpu.CompilerParams(needs_layout_passes=False),
        scratch_types=[
            pltpu.VMEM((n_nodes,), jnp.float32),
            pltpu.VMEM((e_w,), jnp.int32),
            pltpu.VMEM((e_w,), jnp.int32),
            pltpu.VMEM((n_pad,), jnp.float32),
        ],
    )


def _dinv_from_partials(degp, n):
    deg = jnp.sum(degp[:, :n], axis=0) + 1.0
    return lax.rsqrt(deg)


def _split_store(out_ref, r, half):
    out_ref[0] = r[:, :half]
    out_ref[1] = r[:, half:]


def _tc_first_body(half, n, degp_ref, x_ref, w_ref, hs_ref):
    dinv = _dinv_from_partials(degp_ref[...], n)
    h = jnp.dot(x_ref[...], w_ref[...], preferred_element_type=jnp.float32)
    _split_store(hs_ref, h * dinv[:, None], half)


def _tc_mid_body(half, n, split, degp_ref, agg_ref, hsp_ref, b_ref, w_ref,
                 out_ref):
    dinv = _dinv_from_partials(degp_ref[...], n)
    a = (jnp.concatenate([agg_ref[0][:n], agg_ref[1][:n]], axis=-1)
         + jnp.concatenate([hsp_ref[0], hsp_ref[1]], axis=-1))
    h = jnp.maximum(a * dinv[:, None] + b_ref[...], 0.0)
    r = (jnp.dot(h, w_ref[...], preferred_element_type=jnp.float32)
         * dinv[:, None])
    if split:
        _split_store(out_ref, r, half)
    else:
        out_ref[...] = r


def _tc_final_body(n, degp_ref, accs_ref, z_ref, b_ref, out_ref):
    dinv = _dinv_from_partials(degp_ref[...], n)
    a = jnp.sum(accs_ref[...][:, :n], axis=0) + z_ref[...][:, 0]
    out_ref[...] = (dinv * a)[:, None] + b_ref[...]


def kernel(x, edge_index, W1, b1, W2, b2, W3, b3):
    n_nodes, in_dim = x.shape
    hid = W1.shape[1]
    half = hid // 2
    n_edges = edge_index.shape[1]
    n_chunks, e_pad, n_pad = _pad_sizes(n_edges, n_nodes)
    e_w = e_pad // NW
    pad = e_pad - n_edges

    src = jnp.concatenate([edge_index[0], jnp.zeros((pad,), jnp.int32)])
    dst = jnp.concatenate(
        [edge_index[1], jnp.full((pad,), n_nodes, jnp.int32)])
    dst2d = dst.reshape(e_pad // CHUNK, CHUNK)
    src1d = src.reshape(NW, e_w)
    dst1d = dst.reshape(NW, e_w)
    zeros_blk = jnp.zeros((n_pad // NS, half), jnp.float32)

    degree_k = _make_degree_kernel(n_pad, e_w)
    agg_k = _make_agg_kernel(n_pad, n_chunks, half)
    scalar_k = _make_scalar_agg_kernel(n_nodes, n_pad, e_w)

    degp = degree_k(dst1d)

    hs1 = pl.pallas_call(
        functools.partial(_tc_first_body, half, n_nodes),
        out_shape=jax.ShapeDtypeStruct((NC, n_nodes, half), jnp.float32),
    )(degp, x, W1)
    agg1 = agg_k(hs1, src, dst2d, zeros_blk)
    hs2 = pl.pallas_call(
        functools.partial(_tc_mid_body, half, n_nodes, True),
        out_shape=jax.ShapeDtypeStruct((NC, n_nodes, half), jnp.float32),
    )(degp, agg1, hs1, b1.reshape(1, hid), W2)
    agg2 = agg_k(hs2, src, dst2d, zeros_blk)
    z = pl.pallas_call(
        functools.partial(_tc_mid_body, half, n_nodes, False),
        out_shape=jax.ShapeDtypeStruct((n_nodes, 1), jnp.float32),
    )(degp, agg2, hs2, b2.reshape(1, hid), W3)
    accs = scalar_k(z.reshape(n_nodes), src1d, dst1d)
    out = pl.pallas_call(
        functools.partial(_tc_final_body, n_nodes),
        out_shape=jax.ShapeDtypeStruct((n_nodes, 1), jnp.float32),
    )(degp, accs, z, b3.reshape(1, 1))
    return out
